# baseline (speedup 1.0000x reference)
import functools

import jax
import jax.numpy as jnp
from jax.experimental import pallas as pl
from jax.experimental.pallas import tpu as pltpu


def _staged_chains(xs, bases, lo, hi, wvu, wa, h, sub):

    x8s = [x.astype(jnp.float8_e4m3fn) for x in xs]
    gs = [jnp.dot(x8, wvu, preferred_element_type=jnp.float32)
          for x8 in x8s]
    a2s = [jnp.tanh(g[:, :h] * (1.0 / 64.0)) * (
        jnp.tanh(g[:, h:] * (1.0 / 128.0)) + 1.0) for g in gs]
    scs = [jnp.dot(a2.astype(jnp.bfloat16), wa,
                   preferred_element_type=jnp.float32) for a2 in a2s]

    iot = jax.lax.broadcasted_iota(jnp.int32, (sub, 1), 0)
    ohs = [((base + iot) >= lo) & ((base + iot) < hi) for base in bases]
    es = [jnp.where(oh, jnp.exp(sc), 0.0)
          for oh, sc in zip(ohs, scs)]

    pools = [jax.lax.dot_general(
        e, x, (((0,), (0,)), ((), ())),
        preferred_element_type=jnp.float32) for e, x in zip(es, xs)]
    dens = [jax.lax.dot_general(
        e, jnp.ones((sub, 1), jnp.float32), (((0,), (0,)), ((), ())),
        preferred_element_type=jnp.float32) for e in es]
    return pools, dens


def _prism_kernel(x_ref, lo_ref, hi_ref, wvu_ref, wa_ref, wc_ref, bc_ref,
                  out_ref, acc_ref, den_ref, *, nblocks, blk, nsub, h):
    i = pl.program_id(0)

    @pl.when(i == 0)
    def _init():
        acc_ref[...] = jnp.zeros_like(acc_ref)
        den_ref[...] = jnp.zeros_like(den_ref)

    sub = blk // nsub
    lo = lo_ref[...]
    hi = hi_ref[...]
    wvu = wvu_ref[...]
    wa = wa_ref[...]
    xs = [x_ref[j * sub:(j + 1) * sub] for j in range(nsub)]
    bases = [i * blk + j * sub for j in range(nsub)]
    pools, dens = _staged_chains(xs, bases, lo, hi, wvu, wa, h, sub)
    acc_ref[...] += sum(pools)
    den_ref[...] += sum(dens)

    @pl.when(i == nblocks - 1)
    def _finish():
        logits = jnp.dot(acc_ref[...], wc_ref[...],
                         preferred_element_type=jnp.float32)
        out_ref[...] = logits / den_ref[...] + bc_ref[...]


def kernel(flat, cu_seqlens, Wv, Wu, w_attn, Wc, bc):
    n, d = flat.shape
    h = Wv.shape[1]
    nseg = cu_seqlens.shape[0] - 1
    c = Wc.shape[1]
    blk = 4096
    nsub = 4
    nblocks = n // blk

    wvu = (jnp.concatenate([Wv, Wu], axis=1) * 64.0).astype(
        jnp.float8_e4m3fn)
    cu = cu_seqlens.astype(jnp.int32)
    lo = cu[:-1].reshape(1, nseg)
    hi = cu[1:].reshape(1, nseg)
    bc2 = bc.reshape(1, c)

    grid_kernel = functools.partial(
        _prism_kernel, nblocks=nblocks, blk=blk, nsub=nsub, h=h)

    return pl.pallas_call(
        grid_kernel,
        grid=(nblocks,),
        in_specs=[
            pl.BlockSpec((blk, d), lambda i: (i, 0)),
            pl.BlockSpec((1, nseg), lambda i: (0, 0)),
            pl.BlockSpec((1, nseg), lambda i: (0, 0)),
            pl.BlockSpec((d, 2 * h), lambda i: (0, 0)),
            pl.BlockSpec((h, 1), lambda i: (0, 0)),
            pl.BlockSpec((d, c), lambda i: (0, 0)),
            pl.BlockSpec((1, c), lambda i: (0, 0)),
        ],
        out_specs=pl.BlockSpec((nseg, c), lambda i: (0, 0)),
        out_shape=jax.ShapeDtypeStruct((nseg, c), jnp.float32),
        scratch_shapes=[
            pltpu.VMEM((nseg, d), jnp.float32),
            pltpu.VMEM((nseg, 1), jnp.float32),
        ],
        compiler_params=pltpu.CompilerParams(
            dimension_semantics=("arbitrary",)),
    )(flat, lo, hi, wvu, (0.5 * w_attn).astype(jnp.bfloat16), Wc, bc2)


# speedup vs baseline: 1.0062x; 1.0062x over previous
import functools

import jax
import jax.numpy as jnp
from jax.experimental import pallas as pl
from jax.experimental.pallas import tpu as pltpu


def _staged_chains(xs, bases, lo, hi, wvu, wa, h, sub):
    xbs = [x.astype(jnp.bfloat16) for x in xs]
    x8s = [x.astype(jnp.float8_e4m3fn) for x in xs]
    gs = [jnp.dot(x8, wvu, preferred_element_type=jnp.float32)
          for x8 in x8s]
    a2s = [jnp.tanh(g[:, :h] * (1.0 / 64.0)) * (
        jnp.tanh(g[:, h:] * (1.0 / 128.0)) + 1.0) for g in gs]
    scs = [jnp.dot(a2.astype(jnp.bfloat16), wa,
                   preferred_element_type=jnp.float32) for a2 in a2s]

    iot = jax.lax.broadcasted_iota(jnp.int32, (sub, 1), 0)
    ohs = [((base + iot) >= lo) & ((base + iot) < hi) for base in bases]
    es = [jnp.where(oh, jnp.exp(sc), 0.0)
          for oh, sc in zip(ohs, scs)]

    pools = [jax.lax.dot_general(
        e.astype(jnp.bfloat16), xb, (((0,), (0,)), ((), ())),
        preferred_element_type=jnp.float32) for e, xb in zip(es, xbs)]
    dens = [jax.lax.dot_general(
        e, jnp.ones((sub, 1), jnp.float32), (((0,), (0,)), ((), ())),
        preferred_element_type=jnp.float32) for e in es]
    return pools, dens


def _prism_kernel(x_ref, lo_ref, hi_ref, wvu_ref, wa_ref, wc_ref, bc_ref,
                  out_ref, acc_ref, den_ref, *, nblocks, blk, nsub, h):
    i = pl.program_id(0)

    @pl.when(i == 0)
    def _init():
        acc_ref[...] = jnp.zeros_like(acc_ref)
        den_ref[...] = jnp.zeros_like(den_ref)

    sub = blk // nsub
    lo = lo_ref[...]
    hi = hi_ref[...]
    wvu = wvu_ref[...]
    wa = wa_ref[...]
    xs = [x_ref[j * sub:(j + 1) * sub] for j in range(nsub)]
    bases = [i * blk + j * sub for j in range(nsub)]
    pools, dens = _staged_chains(xs, bases, lo, hi, wvu, wa, h, sub)
    acc_ref[...] += sum(pools)
    den_ref[...] += sum(dens)

    @pl.when(i == nblocks - 1)
    def _finish():
        logits = jnp.dot(acc_ref[...], wc_ref[...],
                         preferred_element_type=jnp.float32)
        out_ref[...] = logits / den_ref[...] + bc_ref[...]


def kernel(flat, cu_seqlens, Wv, Wu, w_attn, Wc, bc):
    n, d = flat.shape
    h = Wv.shape[1]
    nseg = cu_seqlens.shape[0] - 1
    c = Wc.shape[1]
    blk = 4096
    nsub = 4
    nblocks = n // blk

    wvu = (jnp.concatenate([Wv, Wu], axis=1) * 64.0).astype(
        jnp.float8_e4m3fn)
    cu = cu_seqlens.astype(jnp.int32)
    lo = cu[:-1].reshape(1, nseg)
    hi = cu[1:].reshape(1, nseg)
    bc2 = bc.reshape(1, c)

    grid_kernel = functools.partial(
        _prism_kernel, nblocks=nblocks, blk=blk, nsub=nsub, h=h)

    return pl.pallas_call(
        grid_kernel,
        grid=(nblocks,),
        in_specs=[
            pl.BlockSpec((blk, d), lambda i: (i, 0)),
            pl.BlockSpec((1, nseg), lambda i: (0, 0)),
            pl.BlockSpec((1, nseg), lambda i: (0, 0)),
            pl.BlockSpec((d, 2 * h), lambda i: (0, 0)),
            pl.BlockSpec((h, 1), lambda i: (0, 0)),
            pl.BlockSpec((d, c), lambda i: (0, 0)),
            pl.BlockSpec((1, c), lambda i: (0, 0)),
        ],
        out_specs=pl.BlockSpec((nseg, c), lambda i: (0, 0)),
        out_shape=jax.ShapeDtypeStruct((nseg, c), jnp.float32),
        scratch_shapes=[
            pltpu.VMEM((nseg, d), jnp.float32),
            pltpu.VMEM((nseg, 1), jnp.float32),
        ],
        compiler_params=pltpu.CompilerParams(
            dimension_semantics=("arbitrary",)),
    )(flat, lo, hi, wvu, (0.5 * w_attn).astype(jnp.bfloat16), Wc, bc2)
